# baseline (speedup 1.0000x reference)
import functools
import jax
import jax.numpy as jnp
from jax import lax
from jax.experimental import pallas as pl
from jax.experimental.pallas import tpu as pltpu
from jax.experimental.pallas import tpu_sc as plsc

_E = 64
_K = 8
_TEMP = 1.0
_EPS = 1e-6
_TILE = 1024
_NC = 2
_NS = 16
_NW = _NC * _NS
_L = 16


def _scale_kernel(h_ref, out_ref):
    out_ref[...] = jnp.full((8, 128), jnp.maximum(jnp.max(jnp.abs(h_ref[...])), _EPS))


def _routing_kernel(logits_hbm, scale_hbm, w_hbm, idx_hbm, logits_v, scale_v, w_v, idx_v):
    n = logits_hbm.shape[0]
    tpw = n // _NW
    wid = lax.axis_index("s") * _NC + lax.axis_index("c")
    base = wid * tpw
    pltpu.sync_copy(logits_hbm.at[pl.ds(base, tpw)], logits_v)
    pltpu.sync_copy(scale_hbm, scale_v)
    inv = (1.0 / scale_v[...]) * (1.0 / max(_TEMP, _EPS))
    lane = lax.iota(jnp.int32, _L)
    lo8 = lane < 8
    zero16 = jnp.zeros((_L,), jnp.float32)

    def merge(ak, av, bk, bv):
        mk = jnp.where(lo8, ak, lax.rev(bk, (0,)))
        mv = jnp.where(lo8, av, lax.rev(bv, (0,)))
        return plsc.sort_key_val(mk, mv, descending=True)

    @plsc.parallel_loop(0, tpw, 1)
    def body(t):
        ks, vs = [], []
        mx = None
        for j in range(4):
            lj = logits_v[t, pl.ds(16 * j, _L)] * inv
            mx = lj if mx is None else jnp.maximum(mx, lj)
            kj, vj = plsc.sort_key_val(lj, lane + 16 * j, descending=True)
            ks.append(kj)
            vs.append(vj)
        m = lax.reduce_max(mx, axes=(0,))
        k01, v01 = merge(ks[0], vs[0], ks[1], vs[1])
        k23, v23 = merge(ks[2], vs[2], ks[3], vs[3])
        fk, fv = merge(k01, v01, k23, v23)
        pv = jnp.exp(fk - jnp.full((_L,), m))
        pm = jnp.where(lo8, pv, zero16)
        s = lax.reduce_sum(pm, axes=(0,))
        w8 = pm / jnp.full((_L,), s)
        for j in range(4):
            w_v[t, pl.ds(16 * j, _L)] = zero16
        tvec = jnp.full((_L,), t, jnp.int32)
        plsc.store_scatter(w_v, [tvec, fv], w8, mask=lo8)
        idx_v[t, :] = fv

    pltpu.sync_copy(w_v, w_hbm.at[pl.ds(base, tpw)])
    pltpu.sync_copy(idx_v, idx_hbm.at[pl.ds(base, tpw)])


def _mix_kernel(w_ref, limes_ref, pmix_ref):
    pmix_ref[...] = jnp.dot(w_ref[...], limes_ref[...],
                            preferred_element_type=jnp.float32)


@jax.jit
def kernel(H, LiMEs):
    B, T, D = H.shape
    N = B * T
    Hs = H.reshape(N, D)[:, :_E]

    scale_blk = pl.pallas_call(
        _scale_kernel,
        in_specs=[pl.BlockSpec((N, _E), lambda: (0, 0))],
        out_specs=pl.BlockSpec((8, 128), lambda: (0, 0)),
        out_shape=jax.ShapeDtypeStruct((8, 128), jnp.float32),
    )(Hs)
    scale16 = scale_blk[0, :_L]

    tpw = N // _NW
    mesh = plsc.VectorSubcoreMesh(core_axis_name="c", subcore_axis_name="s")
    routing = functools.partial(
        pl.kernel,
        mesh=mesh,
        out_type=[
            jax.ShapeDtypeStruct((N, _E), jnp.float32),
            jax.ShapeDtypeStruct((N, _L), jnp.int32),
        ],
        scratch_types=[
            pltpu.VMEM((tpw, _E), jnp.float32),
            pltpu.VMEM((_L,), jnp.float32),
            pltpu.VMEM((tpw, _E), jnp.float32),
            pltpu.VMEM((tpw, _L), jnp.int32),
        ],
        compiler_params=pltpu.CompilerParams(needs_layout_passes=False),
    )(_routing_kernel)
    W, idx16 = routing(Hs, scale16)

    grid = (N // _TILE,)
    pmix = pl.pallas_call(
        _mix_kernel,
        grid=grid,
        in_specs=[
            pl.BlockSpec((_TILE, _E), lambda i: (i, 0)),
            pl.BlockSpec((_E, D), lambda i: (0, 0)),
        ],
        out_specs=pl.BlockSpec((_TILE, D), lambda i: (i, 0)),
        out_shape=jax.ShapeDtypeStruct((N, D), jnp.float32),
    )(W, LiMEs)

    return pmix.reshape(B, T, D), idx16[:, :_K].reshape(B, T, _K)


# speedup vs baseline: 1.1161x; 1.1161x over previous
import functools
import jax
import jax.numpy as jnp
from jax import lax
from jax.experimental import pallas as pl
from jax.experimental.pallas import tpu as pltpu
from jax.experimental.pallas import tpu_sc as plsc

_E = 64
_K = 8
_TEMP = 1.0
_EPS = 1e-6
_TILE = 512
_NC = 2
_NS = 16
_NW = _NC * _NS
_L = 16


def _topk_kernel(logits_hbm, idx_hbm, logits_v, idx_v):
    n = logits_hbm.shape[0]
    tpw = n // _NW
    wid = lax.axis_index("s") * _NC + lax.axis_index("c")
    base = wid * tpw
    pltpu.sync_copy(logits_hbm.at[pl.ds(base, tpw)], logits_v)
    lane = lax.iota(jnp.int32, _L)
    lo8 = lane < 8

    def merge(ak, av, bk, bv):
        mk = jnp.where(lo8, ak, lax.rev(bk, (0,)))
        mv = jnp.where(lo8, av, lax.rev(bv, (0,)))
        return plsc.sort_key_val(mk, mv, descending=True)

    @plsc.parallel_loop(0, tpw, 1)
    def body(t):
        ks, vs = [], []
        for j in range(4):
            lj = logits_v[t, pl.ds(16 * j, _L)]
            kj, vj = plsc.sort_key_val(lj, lane + 16 * j, descending=True)
            ks.append(kj)
            vs.append(vj)
        k01, v01 = merge(ks[0], vs[0], ks[1], vs[1])
        k23, v23 = merge(ks[2], vs[2], ks[3], vs[3])
        _, fv = merge(k01, v01, k23, v23)
        idx_v[t, :] = fv

    pltpu.sync_copy(idx_v, idx_hbm.at[pl.ds(base, tpw)])


def _mix_kernel(hall_ref, limes_ref, idxkn_ref, pmix_ref, scale_ref):
    i = pl.program_id(0)

    @pl.when(i == 0)
    def _():
        scale_ref[0, 0] = jnp.maximum(jnp.max(jnp.abs(hall_ref[...])), _EPS)

    inv = (1.0 / scale_ref[0, 0]) * (1.0 / max(_TEMP, _EPS))
    logits = hall_ref[:, pl.ds(i * _TILE, _TILE)] * inv
    m = jnp.max(logits, axis=0, keepdims=True)
    p = jnp.exp(logits - m)
    iota = jax.lax.broadcasted_iota(jnp.int32, logits.shape, 0)
    idxb = idxkn_ref[...]
    mask = jnp.zeros(logits.shape, jnp.bool_)
    for k in range(_K):
        mask = jnp.logical_or(mask, iota == idxb[k:k + 1, :])
    wm = jnp.where(mask, p, 0.0)
    wsum = jnp.sum(wm, axis=0, keepdims=True)
    w = wm / wsum
    pmix_ref[...] = jax.lax.dot_general(
        w, limes_ref[...], (((0,), (0,)), ((), ())),
        preferred_element_type=jnp.float32,
    )


@jax.jit
def kernel(H, LiMEs):
    B, T, D = H.shape
    N = B * T
    Hs = H.reshape(N, D)[:, :_E]

    tpw = N // _NW
    mesh = plsc.VectorSubcoreMesh(core_axis_name="c", subcore_axis_name="s")
    topk = functools.partial(
        pl.kernel,
        mesh=mesh,
        out_type=jax.ShapeDtypeStruct((N, _L), jnp.int32),
        scratch_types=[
            pltpu.VMEM((tpw, _E), jnp.float32),
            pltpu.VMEM((tpw, _L), jnp.int32),
        ],
        compiler_params=pltpu.CompilerParams(needs_layout_passes=False),
    )(_topk_kernel)
    idx16 = topk(Hs)
    idx8 = idx16[:, :_K]
    idxKN = idx8.T

    grid = (N // _TILE,)
    pmix = pl.pallas_call(
        _mix_kernel,
        grid=grid,
        in_specs=[
            pl.BlockSpec((_E, N), lambda i: (0, 0)),
            pl.BlockSpec((_E, D), lambda i: (0, 0)),
            pl.BlockSpec((_K, _TILE), lambda i: (0, i)),
        ],
        out_specs=pl.BlockSpec((_TILE, D), lambda i: (i, 0)),
        out_shape=jax.ShapeDtypeStruct((N, D), jnp.float32),
        scratch_shapes=[pltpu.SMEM((1, 1), jnp.float32)],
    )(Hs.T, LiMEs, idxKN)

    return pmix.reshape(B, T, D), idx8.reshape(B, T, _K)
